# Initial kernel scaffold; baseline (speedup 1.0000x reference)
import jax
import jax.numpy as jnp
from jax.experimental import pallas as pl


def kernel(feats, points, neighbors, ln_g, ln_b, W1, b1, W2, b2, Wd, bd, Wg1, bg1, Wg2, bg2):
    raise NotImplementedError("write your pallas kernel here")


# speedup vs baseline: 2.4346x; 2.4346x over previous
import functools

import jax
import jax.numpy as jnp
from jax import lax
from jax.experimental import pallas as pl
from jax.experimental.pallas import tpu as pltpu
from jax.experimental.pallas import tpu_sc as plsc


def _ln_body(f_ref, g_ref, b_ref, o_ref):
    f = f_ref[...]
    mu = jnp.mean(f, axis=-1, keepdims=True)
    var = jnp.mean((f - mu) * (f - mu), axis=-1, keepdims=True)
    o_ref[...] = (f - mu) * lax.rsqrt(var + 1e-5) * g_ref[...] + b_ref[...]


def _layernorm_table(feats, ln_g, ln_b, n_pad, block_rows=1000):
    n, d = feats.shape
    grid = n // block_rows
    return pl.pallas_call(
        _ln_body,
        grid=(grid,),
        in_specs=[
            pl.BlockSpec((block_rows, d), lambda i: (i, 0)),
            pl.BlockSpec((1, d), lambda i: (0, 0)),
            pl.BlockSpec((1, d), lambda i: (0, 0)),
        ],
        out_specs=pl.BlockSpec((block_rows, d), lambda i: (i, 0)),
        out_shape=jax.ShapeDtypeStruct((n_pad, d), jnp.float32),
    )(feats, ln_g.reshape(1, d), ln_b.reshape(1, d))


def _sc_dist2(table, nbr_flat, pts_flat, n_real, k, cpw, cb):
    np_, dc = table.shape
    n3 = pts_flat.shape[0]
    info = plsc.get_sparse_core_info()
    nc, ns = info.num_cores, info.num_subcores
    assert np_ == nc * ns * cpw and cpw % cb == 0
    nb = cpw // cb
    rows_b = cb * k
    dch = dc // 16

    mesh = plsc.VectorSubcoreMesh(core_axis_name="c", subcore_axis_name="s")

    @functools.partial(
        pl.kernel,
        out_type=(
            jax.ShapeDtypeStruct((np_, k), jnp.float32),
            jax.ShapeDtypeStruct((np_, k * 16), jnp.float32),
        ),
        mesh=mesh,
        compiler_params=pltpu.CompilerParams(needs_layout_passes=False),
        scratch_types=[
            pltpu.VMEM((cpw * k,), jnp.int32),
            pltpu.VMEM((n3,), jnp.float32),
            pltpu.VMEM((2, rows_b, dc), jnp.float32),
            pltpu.VMEM((2, cb, dc), jnp.float32),
            pltpu.VMEM((2, cb, k), jnp.float32),
            pltpu.VMEM((2, cb, k * 16), jnp.float32),
            pltpu.SemaphoreType.DMA((2,)),
            pltpu.SemaphoreType.DMA((2,)),
        ],
    )
    def sc_kernel(tab_hbm, nbr_hbm, pts_hbm, d2g_hbm, d2f_hbm,
                  idx_v, pts_v, g_v, c_v, og_v, of_v, sems, osems):
        wid = lax.axis_index("s") * nc + lax.axis_index("c")
        base = wid * cpw

        pltpu.sync_copy(nbr_hbm.at[pl.ds(base * k, cpw * k)], idx_v)
        pltpu.sync_copy(pts_hbm, pts_v)

        def clamp_body(i, _):
            row = idx_v[pl.ds(i * 16, 16)]
            idx_v[pl.ds(i * 16, 16)] = jnp.minimum(
                jnp.maximum(row, 0), n_real - 1)
            return 0
        lax.fori_loop(0, cpw * k // 16, clamp_body, 0)

        def issue(b, slot):
            pltpu.async_copy(
                tab_hbm.at[idx_v.at[pl.ds(b * rows_b, rows_b)]],
                g_v.at[slot], sems.at[slot])
            pltpu.async_copy(
                tab_hbm.at[pl.ds(base + b * cb, cb)],
                c_v.at[slot], sems.at[slot])

        def drain(b, slot):
            pltpu.make_async_copy(
                tab_hbm.at[idx_v.at[pl.ds(b * rows_b, rows_b)]],
                g_v.at[slot], sems.at[slot]).wait()
            pltpu.make_async_copy(
                tab_hbm.at[pl.ds(base + b * cb, cb)],
                c_v.at[slot], sems.at[slot]).wait()

        def issue_out(b, slot):
            pltpu.async_copy(og_v.at[slot],
                             d2g_hbm.at[pl.ds(base + b * cb, cb)],
                             osems.at[slot])
            pltpu.async_copy(of_v.at[slot],
                             d2f_hbm.at[pl.ds(base + b * cb, cb)],
                             osems.at[slot])

        def drain_out(b, slot):
            pltpu.make_async_copy(og_v.at[slot],
                                  d2g_hbm.at[pl.ds(base + b * cb, cb)],
                                  osems.at[slot]).wait()
            pltpu.make_async_copy(of_v.at[slot],
                                  d2f_hbm.at[pl.ds(base + b * cb, cb)],
                                  osems.at[slot]).wait()

        issue(0, 0)
        issue(1, 1)

        zeros16 = jnp.zeros((16,), jnp.int32)

        def compute_batch(b, slot):
            gs = g_v.at[slot]
            cs = c_v.at[slot]
            ogs = og_v.at[slot]
            ofs = of_v.at[slot]

            def c_body(c, _):
                row = b * cb + c
                i_eff = jnp.minimum(base + row, n_real - 1)
                jvec = idx_v[pl.ds(row * k, 16)] * 3
                acc_g = jnp.zeros((16,), jnp.float32)
                for axis in range(3):
                    pj = plsc.load_gather(pts_v, [jvec + axis])
                    pi = plsc.load_gather(pts_v,
                                          [zeros16 + (i_eff * 3 + axis)])
                    dd = pj - pi
                    acc_g = acc_g + dd * dd
                ogs[c, :] = acc_g
                cvec = [cs[c, pl.ds(t * 16, 16)] for t in range(dch)]
                for kk in range(k):
                    a0 = jnp.zeros((16,), jnp.float32)
                    a1 = jnp.zeros((16,), jnp.float32)
                    for t in range(dch):
                        g = gs[c * k + kk, pl.ds(t * 16, 16)]
                        dd = g - cvec[t]
                        if t % 2 == 0:
                            a0 = a0 + dd * dd
                        else:
                            a1 = a1 + dd * dd
                    ofs[c, pl.ds(kk * 16, 16)] = a0 + a1
                return 0
            lax.fori_loop(0, cb, c_body, 0)

        def outer(i, _):
            for s in range(2):
                b = i * 2 + s
                drain(b, s)

                @pl.when(b >= 2)
                def _():
                    drain_out(b - 2, s)

                @pl.when(b + 2 < nb)
                def _():
                    issue(b + 2, s)
                compute_batch(b, s)
                issue_out(b, s)
            return 0
        lax.fori_loop(0, nb // 2, outer, 0)
        drain_out(nb - 2, 0)
        drain_out(nb - 1, 1)

    return sc_kernel(table, nbr_flat, pts_flat)


def _gelu(x):
    return 0.5 * x * (1.0 + lax.erf(x * 0.7071067811865476))


def _mlp_body(x_ref, d2g_ref, d2f_ref, nbr_ref, s_ref,
              w1x_ref, w1m_ref, w1v_ref, w1f_ref, b1_ref,
              w2_ref, b2_ref, wd_ref, bd_ref,
              wg1_ref, wg1d_ref, bg1_ref, wg2_ref, bg2_ref,
              diff_ref, gw_ref, *, n_total, d_model):
    nbr = nbr_ref[...]
    valid = ((nbr >= 0) & (nbr < n_total)).astype(jnp.float32)
    denom = jnp.maximum(jnp.sum(valid, axis=1, keepdims=True), 1.0)
    seg = s_ref[...]
    mm = lambda a, b: jax.lax.dot_general(
        a, b, (((1,), (0,)), ((), ())),
        precision=lax.Precision.HIGHEST, preferred_element_type=jnp.float32)
    dist = jnp.sqrt(jnp.maximum(d2g_ref[...], 0.0))
    fdist = jnp.sqrt(jnp.maximum(mm(d2f_ref[...], seg), 0.0))
    mean_dist = jnp.sum(dist * valid, axis=1, keepdims=True) / denom
    e2 = jnp.sum(dist * dist * valid, axis=1, keepdims=True) / denom
    dist_var = e2 - mean_dist * mean_dist
    feat_var = jnp.sum(fdist * valid, axis=1, keepdims=True) / denom

    x = x_ref[...]
    pre1 = (mm(x, w1x_ref[...])
            + mean_dist * w1m_ref[...] + dist_var * w1v_ref[...]
            + feat_var * w1f_ref[...] + b1_ref[...])
    h1 = _gelu(pre1)
    hidden = _gelu(mm(h1, w2_ref[...]) + b2_ref[...])
    dlogit = jnp.sum(hidden * wd_ref[...], axis=1, keepdims=True) + bd_ref[0, 0]
    difficulty = jax.nn.sigmoid(dlogit)
    g = _gelu(mm(hidden, wg1_ref[...]) + difficulty * wg1d_ref[...]
              + bg1_ref[...])
    glogit = jnp.sum(g * wg2_ref[...], axis=1, keepdims=True) + bg2_ref[0, 0]
    diff_ref[...] = difficulty
    gw_ref[...] = jax.nn.sigmoid(glogit + dlogit)


def _mlp(table, d2g, d2f, nbr, W1, b1, W2, b2, Wd, bd, Wg1, bg1, Wg2, bg2,
         n, d, block_rows=1000):
    k = nbr.shape[1]
    h = W2.shape[0]
    grid = n // block_rows
    w1x = W1[:d]
    w1m = W1[d:d + 1]
    w1v = W1[d + 1:d + 2]
    w1f = W1[d + 2:d + 3]
    wg1h = Wg1[:h]
    wg1d = Wg1[h:h + 1]
    seg = (jnp.arange(k * 16)[:, None] // 16
           == jnp.arange(k)[None, :]).astype(jnp.float32)

    full = lambda shape: pl.BlockSpec(shape, lambda i: tuple(0 for _ in shape))
    row_spec = lambda w: pl.BlockSpec((block_rows, w), lambda i: (i, 0))
    return pl.pallas_call(
        functools.partial(_mlp_body, n_total=n, d_model=d),
        grid=(grid,),
        in_specs=[
            row_spec(d), row_spec(k), row_spec(k * 16), row_spec(k),
            full((k * 16, k)),
            full((d, h)), full((1, h)), full((1, h)), full((1, h)), full((1, h)),
            full((h, h)), full((1, h)), full((1, h)), full((1, 1)),
            full((h, h)), full((1, h)), full((1, h)), full((1, h)), full((1, 1)),
        ],
        out_specs=[row_spec(1), row_spec(1)],
        out_shape=[
            jax.ShapeDtypeStruct((n, 1), jnp.float32),
            jax.ShapeDtypeStruct((n, 1), jnp.float32),
        ],
    )(table, d2g, d2f, nbr, seg,
      w1x, w1m, w1v, w1f, b1.reshape(1, h),
      W2, b2.reshape(1, h), Wd.reshape(1, h), bd.reshape(1, 1),
      wg1h, wg1d, bg1.reshape(1, h), Wg2.reshape(1, h), bg2.reshape(1, 1))


def kernel(feats, points, neighbors, ln_g, ln_b, W1, b1, W2, b2, Wd, bd,
           Wg1, bg1, Wg2, bg2):
    n, d = feats.shape
    k = neighbors.shape[1]
    nw = 32
    cpw = 320
    np_ = nw * cpw
    cb = 8

    table = _layernorm_table(feats, ln_g, ln_b, np_)
    nbr_flat = jnp.pad(neighbors.astype(jnp.int32),
                       ((0, np_ - n), (0, 0))).reshape(-1)
    d2g_p, d2f_p = _sc_dist2(table, nbr_flat, points.reshape(-1),
                             n, k, cpw, cb)
    return _mlp(table, d2g_p, d2f_p, neighbors.astype(jnp.int32),
                W1, b1, W2, b2, Wd, bd, Wg1, bg1, Wg2, bg2, n, d)
